# baseline (speedup 1.0000x reference)
import functools

import jax
import jax.numpy as jnp
from jax import lax
from jax.experimental import pallas as pl
from jax.experimental.pallas import tpu as pltpu
from jax.experimental.pallas import tpu_sc as plsc

N = 10000
E = 320000
D = 128
G = 128

NPAD = 10240
CH = 128
NW = 32
EPAD = 327680
NITER = EPAD // (NW * CH)
NPH = 2
CPH = NITER // NPH

ROWS_BLK = 1000
GRID = N // ROWS_BLK


def _agg_body(h_hbm, src_hbm, dst_hbm, zeros_hbm, out_hbm,
              src_v0, dst_v0, src_v1, dst_v1, rows_a, rows_b, acc_sh, sem_a, sem_b):
    cid = lax.axis_index("c")
    sid = lax.axis_index("s")
    wid = sid * 2 + cid

    pltpu.sync_copy(zeros_hbm.at[pl.ds(sid * 640, 640)],
                    acc_sh.at[pl.ds(sid * 640, 640)])
    plsc.subcore_barrier()

    base0 = wid * CH

    pltpu.sync_copy(src_hbm.at[pl.ds(base0, CH)], src_v0)
    pltpu.sync_copy(dst_hbm.at[pl.ds(base0, CH)], dst_v0)
    pltpu.async_copy(h_hbm.at[src_v0], rows_a, sem_a)

    def body(i, carry):
        k1 = base0 + (2 * i + 1) * NW * CH
        k2 = base0 + jnp.minimum(2 * i + 2, NITER - 1) * NW * CH
        pltpu.sync_copy(src_hbm.at[pl.ds(k1, CH)], src_v1)
        pltpu.sync_copy(dst_hbm.at[pl.ds(k1, CH)], dst_v1)
        pltpu.async_copy(h_hbm.at[src_v1], rows_b, sem_b)
        pltpu.make_async_copy(h_hbm.at[src_v0], rows_a, sem_a).wait()
        pltpu.sync_copy(rows_a, acc_sh.at[dst_v0], add=True)
        pltpu.sync_copy(src_hbm.at[pl.ds(k2, CH)], src_v0)
        pltpu.sync_copy(dst_hbm.at[pl.ds(k2, CH)], dst_v0)
        pltpu.async_copy(h_hbm.at[src_v0], rows_a, sem_a)
        pltpu.make_async_copy(h_hbm.at[src_v1], rows_b, sem_b).wait()
        pltpu.sync_copy(rows_b, acc_sh.at[dst_v1], add=True)
        return carry

    lax.fori_loop(0, NITER // 2, body, 0)
    pltpu.make_async_copy(h_hbm.at[src_v0], rows_a, sem_a).wait()
    plsc.subcore_barrier()

    wbase = jnp.minimum(sid * 632, N - 632)
    pltpu.sync_copy(acc_sh.at[pl.ds(wbase, 632)],
                    out_hbm.at[cid, pl.ds(wbase, 632)])


_agg_call = pl.kernel(
    _agg_body,
    out_type=jax.ShapeDtypeStruct((2, N, D), jnp.float32),
    mesh=plsc.VectorSubcoreMesh(core_axis_name="c", subcore_axis_name="s"),
    scratch_types=[
        pltpu.VMEM((CH,), jnp.int32),
        pltpu.VMEM((CH,), jnp.int32),
        pltpu.VMEM((CH,), jnp.int32),
        pltpu.VMEM((CH,), jnp.int32),
        pltpu.VMEM((CH, D), jnp.float32),
        pltpu.VMEM((CH, D), jnp.float32),
        pltpu.VMEM_SHARED((NPAD, D), jnp.float32),
        pltpu.SemaphoreType.DMA,
        pltpu.SemaphoreType.DMA,
    ],
)


def _proj_body(x_ref, w_ref, o_ref):
    o_ref[...] = jnp.dot(x_ref[...], w_ref[...],
                         preferred_element_type=jnp.float32)


def _mlp_body(agg_ref, h_ref, w1_ref, b1_ref, w2_ref, b2_ref, o_ref):
    z = agg_ref[0] + agg_ref[1] + h_ref[...]
    z = jnp.maximum(
        jnp.dot(z, w1_ref[...], preferred_element_type=jnp.float32)
        + b1_ref[...], 0.0)
    z = jnp.maximum(
        jnp.dot(z, w2_ref[...], preferred_element_type=jnp.float32)
        + b2_ref[...], 0.0)
    o_ref[...] = z


def _final_body(agg_ref, h_ref, w1_ref, b1_ref, w2_ref, b2_ref, batch_ref,
                o_ref):
    z = agg_ref[0] + agg_ref[1] + h_ref[...]
    z = jnp.maximum(
        jnp.dot(z, w1_ref[...], preferred_element_type=jnp.float32)
        + b1_ref[...], 0.0)
    z = jnp.maximum(
        jnp.dot(z, w2_ref[...], preferred_element_type=jnp.float32)
        + b2_ref[...], 0.0)
    b = batch_ref[0, 0, :]
    onehot = (b[:, None]
              == lax.broadcasted_iota(jnp.int32, (ROWS_BLK, G), 1)
              ).astype(jnp.float32)
    contrib = lax.dot_general(onehot, z, (((0,), (0,)), ((), ())),
                              preferred_element_type=jnp.float32)

    @pl.when(pl.program_id(0) == 0)
    def _():
        o_ref[...] = jnp.zeros_like(o_ref)

    o_ref[...] += contrib


_W_SPEC = pl.BlockSpec((D, D), lambda i: (0, 0))
_B_SPEC = pl.BlockSpec((1, D), lambda i: (0, 0))
_ROW_SPEC = pl.BlockSpec((ROWS_BLK, D), lambda i: (i, 0))
_AGG_SPEC = pl.BlockSpec((2, ROWS_BLK, D), lambda i: (0, i, 0))

_proj_call = pl.pallas_call(
    _proj_body,
    grid=(GRID,),
    in_specs=[_ROW_SPEC, _W_SPEC],
    out_specs=_ROW_SPEC,
    out_shape=jax.ShapeDtypeStruct((N, D), jnp.float32),
)

_mlp_call = pl.pallas_call(
    _mlp_body,
    grid=(GRID,),
    in_specs=[_AGG_SPEC, _ROW_SPEC, _W_SPEC, _B_SPEC, _W_SPEC, _B_SPEC],
    out_specs=_ROW_SPEC,
    out_shape=jax.ShapeDtypeStruct((N, D), jnp.float32),
)

_final_call = pl.pallas_call(
    _final_body,
    grid=(GRID,),
    in_specs=[_AGG_SPEC, _ROW_SPEC, _W_SPEC, _B_SPEC, _W_SPEC, _B_SPEC,
              pl.BlockSpec((1, 1, ROWS_BLK), lambda i: (i, 0, 0))],
    out_specs=pl.BlockSpec((G, D), lambda i: (0, 0)),
    out_shape=jax.ShapeDtypeStruct((G, D), jnp.float32),
)


def kernel(x, edge_index, batch, W_proj, W1_0, b1_0, W2_0, b2_0,
           W1_1, b1_1, W2_1, b2_1):
    src = jnp.pad(edge_index[0], (0, EPAD - E))
    pad_dst = N + (jnp.arange(EPAD - E, dtype=jnp.int32) % (NPAD - N))
    dst = jnp.concatenate([edge_index[1], pad_dst])
    zeros = jnp.zeros((NPAD, D), jnp.float32)
    batch3d = batch.reshape(GRID, 1, ROWS_BLK)

    h = _proj_call(x, W_proj)

    agg = _agg_call(h, src, dst, zeros)
    h = _mlp_call(agg, h, W1_0, b1_0.reshape(1, D), W2_0, b2_0.reshape(1, D))

    agg = _agg_call(h, src, dst, zeros)
    out = _final_call(agg, h, W1_1, b1_1.reshape(1, D), W2_1,
                      b2_1.reshape(1, D), batch3d)
    return out


# speedup vs baseline: 1.0306x; 1.0306x over previous
import functools

import jax
import jax.numpy as jnp
from jax import lax
from jax.experimental import pallas as pl
from jax.experimental.pallas import tpu as pltpu
from jax.experimental.pallas import tpu_sc as plsc

N = 10000
E = 320000
D = 128
G = 128

NPAD = 10240
CH = 128
NW = 32
EPAD = 327680
NITER = EPAD // (NW * CH)
NPH = 2
CPH = NITER // NPH

ROWS_BLK = 1000
GRID = N // ROWS_BLK


def _agg_body(h_hbm, src_hbm, dst_hbm, zeros_hbm, out_hbm,
              src_v0, dst_v0, src_v1, dst_v1, rows_a, rows_b, acc_sh, sem_a, sem_b):
    cid = lax.axis_index("c")
    sid = lax.axis_index("s")
    wid = sid * 2 + cid

    pltpu.sync_copy(zeros_hbm.at[pl.ds(sid * 640, 640)],
                    acc_sh.at[pl.ds(sid * 640, 640)])
    plsc.subcore_barrier()

    base0 = wid * NITER * CH

    def body(i, carry):
        k0 = base0 + i * CH
        pltpu.sync_copy(src_hbm.at[pl.ds(k0, CH)], src_v0)
        pltpu.sync_copy(dst_hbm.at[pl.ds(k0, CH)], dst_v0)
        pltpu.async_copy(h_hbm.at[src_v0], rows_a, sem_a).wait()
        pltpu.sync_copy(rows_a, acc_sh.at[dst_v0], add=True)
        return carry

    lax.fori_loop(0, NITER, body, 0)
    plsc.subcore_barrier()

    wbase = jnp.minimum(sid * 632, N - 632)
    pltpu.sync_copy(acc_sh.at[pl.ds(wbase, 632)],
                    out_hbm.at[cid, pl.ds(wbase, 632)])


_agg_call = pl.kernel(
    _agg_body,
    out_type=jax.ShapeDtypeStruct((2, N, D), jnp.float32),
    mesh=plsc.VectorSubcoreMesh(core_axis_name="c", subcore_axis_name="s"),
    scratch_types=[
        pltpu.VMEM((CH,), jnp.int32),
        pltpu.VMEM((CH,), jnp.int32),
        pltpu.VMEM((CH,), jnp.int32),
        pltpu.VMEM((CH,), jnp.int32),
        pltpu.VMEM((CH, D), jnp.float32),
        pltpu.VMEM((CH, D), jnp.float32),
        pltpu.VMEM_SHARED((NPAD, D), jnp.float32),
        pltpu.SemaphoreType.DMA,
        pltpu.SemaphoreType.DMA,
    ],
)


def _proj_body(x_ref, w_ref, o_ref):
    o_ref[...] = jnp.dot(x_ref[...], w_ref[...],
                         preferred_element_type=jnp.float32)


def _mlp_body(agg_ref, h_ref, w1_ref, b1_ref, w2_ref, b2_ref, o_ref):
    z = agg_ref[0] + agg_ref[1] + h_ref[...]
    z = jnp.maximum(
        jnp.dot(z, w1_ref[...], preferred_element_type=jnp.float32)
        + b1_ref[...], 0.0)
    z = jnp.maximum(
        jnp.dot(z, w2_ref[...], preferred_element_type=jnp.float32)
        + b2_ref[...], 0.0)
    o_ref[...] = z


def _final_body(agg_ref, h_ref, w1_ref, b1_ref, w2_ref, b2_ref, batch_ref,
                o_ref):
    z = agg_ref[0] + agg_ref[1] + h_ref[...]
    z = jnp.maximum(
        jnp.dot(z, w1_ref[...], preferred_element_type=jnp.float32)
        + b1_ref[...], 0.0)
    z = jnp.maximum(
        jnp.dot(z, w2_ref[...], preferred_element_type=jnp.float32)
        + b2_ref[...], 0.0)
    b = batch_ref[0, 0, :]
    onehot = (b[:, None]
              == lax.broadcasted_iota(jnp.int32, (ROWS_BLK, G), 1)
              ).astype(jnp.float32)
    contrib = lax.dot_general(onehot, z, (((0,), (0,)), ((), ())),
                              preferred_element_type=jnp.float32)

    @pl.when(pl.program_id(0) == 0)
    def _():
        o_ref[...] = jnp.zeros_like(o_ref)

    o_ref[...] += contrib


_W_SPEC = pl.BlockSpec((D, D), lambda i: (0, 0))
_B_SPEC = pl.BlockSpec((1, D), lambda i: (0, 0))
_ROW_SPEC = pl.BlockSpec((ROWS_BLK, D), lambda i: (i, 0))
_AGG_SPEC = pl.BlockSpec((2, ROWS_BLK, D), lambda i: (0, i, 0))

_proj_call = pl.pallas_call(
    _proj_body,
    grid=(GRID,),
    in_specs=[_ROW_SPEC, _W_SPEC],
    out_specs=_ROW_SPEC,
    out_shape=jax.ShapeDtypeStruct((N, D), jnp.float32),
)

_mlp_call = pl.pallas_call(
    _mlp_body,
    grid=(GRID,),
    in_specs=[_AGG_SPEC, _ROW_SPEC, _W_SPEC, _B_SPEC, _W_SPEC, _B_SPEC],
    out_specs=_ROW_SPEC,
    out_shape=jax.ShapeDtypeStruct((N, D), jnp.float32),
)

_final_call = pl.pallas_call(
    _final_body,
    grid=(GRID,),
    in_specs=[_AGG_SPEC, _ROW_SPEC, _W_SPEC, _B_SPEC, _W_SPEC, _B_SPEC,
              pl.BlockSpec((1, 1, ROWS_BLK), lambda i: (i, 0, 0))],
    out_specs=pl.BlockSpec((G, D), lambda i: (0, 0)),
    out_shape=jax.ShapeDtypeStruct((G, D), jnp.float32),
)


def kernel(x, edge_index, batch, W_proj, W1_0, b1_0, W2_0, b2_0,
           W1_1, b1_1, W2_1, b2_1):
    src = jnp.pad(edge_index[0], (0, EPAD - E))
    pad_dst = N + (jnp.arange(EPAD - E, dtype=jnp.int32) % (NPAD - N))
    dst = jnp.concatenate([edge_index[1], pad_dst])
    zeros = jnp.zeros((NPAD, D), jnp.float32)
    batch3d = batch.reshape(GRID, 1, ROWS_BLK)

    h = _proj_call(x, W_proj)

    agg = _agg_call(h, src, dst, zeros)
    h = _mlp_call(agg, h, W1_0, b1_0.reshape(1, D), W2_0, b2_0.reshape(1, D))

    agg = _agg_call(h, src, dst, zeros)
    out = _final_call(agg, h, W1_1, b1_1.reshape(1, D), W2_1,
                      b2_1.reshape(1, D), batch3d)
    return out


# speedup vs baseline: 1.1186x; 1.0854x over previous
import functools

import jax
import jax.numpy as jnp
from jax import lax
from jax.experimental import pallas as pl
from jax.experimental.pallas import tpu as pltpu
from jax.experimental.pallas import tpu_sc as plsc

N = 10000
E = 320000
D = 128
G = 128

NPAD = 10240
CH = 128
NW = 32
EPAD = 327680
NITER = EPAD // (NW * CH)
NPH = 2
CPH = NITER // NPH

ROWS_BLK = 1000
GRID = N // ROWS_BLK


def _agg_body(h_hbm, src_hbm, dst_hbm, zeros_hbm, out_hbm,
              src_v0, dst_v0, src_v1, dst_v1, rows_a, rows_b, acc_sh, sem_a, sem_b):
    cid = lax.axis_index("c")
    sid = lax.axis_index("s")
    wid = sid * 2 + cid

    pltpu.sync_copy(zeros_hbm.at[pl.ds(sid * 640, 640)],
                    acc_sh.at[pl.ds(sid * 640, 640)])
    plsc.subcore_barrier()

    def body(i, carry):
        k0 = (i * NW + wid) * CH
        pltpu.sync_copy(src_hbm.at[pl.ds(k0, CH)], src_v0)
        pltpu.sync_copy(dst_hbm.at[pl.ds(k0, CH)], dst_v0)
        pltpu.async_copy(h_hbm.at[src_v0], rows_a, sem_a).wait()
        pltpu.sync_copy(rows_a, acc_sh.at[dst_v0], add=True)
        return carry

    lax.fori_loop(0, NITER, body, 0)
    plsc.subcore_barrier()

    wbase = jnp.minimum(sid * 632, N - 632)
    pltpu.sync_copy(acc_sh.at[pl.ds(wbase, 632)],
                    out_hbm.at[cid, pl.ds(wbase, 632)])


_agg_call = pl.kernel(
    _agg_body,
    out_type=jax.ShapeDtypeStruct((2, N, D), jnp.float32),
    mesh=plsc.VectorSubcoreMesh(core_axis_name="c", subcore_axis_name="s"),
    scratch_types=[
        pltpu.VMEM((CH,), jnp.int32),
        pltpu.VMEM((CH,), jnp.int32),
        pltpu.VMEM((CH,), jnp.int32),
        pltpu.VMEM((CH,), jnp.int32),
        pltpu.VMEM((CH, D), jnp.float32),
        pltpu.VMEM((CH, D), jnp.float32),
        pltpu.VMEM_SHARED((NPAD, D), jnp.float32),
        pltpu.SemaphoreType.DMA,
        pltpu.SemaphoreType.DMA,
    ],
)


def _proj_body(x_ref, w_ref, o_ref):
    o_ref[...] = jnp.dot(x_ref[...], w_ref[...],
                         preferred_element_type=jnp.float32)


def _mlp_body(agg_ref, h_ref, w1_ref, b1_ref, w2_ref, b2_ref, o_ref):
    z = agg_ref[0] + agg_ref[1] + h_ref[...]
    z = jnp.maximum(
        jnp.dot(z, w1_ref[...], preferred_element_type=jnp.float32)
        + b1_ref[...], 0.0)
    z = jnp.maximum(
        jnp.dot(z, w2_ref[...], preferred_element_type=jnp.float32)
        + b2_ref[...], 0.0)
    o_ref[...] = z


def _final_body(agg_ref, h_ref, w1_ref, b1_ref, w2_ref, b2_ref, batch_ref,
                o_ref):
    z = agg_ref[0] + agg_ref[1] + h_ref[...]
    z = jnp.maximum(
        jnp.dot(z, w1_ref[...], preferred_element_type=jnp.float32)
        + b1_ref[...], 0.0)
    z = jnp.maximum(
        jnp.dot(z, w2_ref[...], preferred_element_type=jnp.float32)
        + b2_ref[...], 0.0)
    b = batch_ref[0, 0, :]
    onehot = (b[:, None]
              == lax.broadcasted_iota(jnp.int32, (ROWS_BLK, G), 1)
              ).astype(jnp.float32)
    contrib = lax.dot_general(onehot, z, (((0,), (0,)), ((), ())),
                              preferred_element_type=jnp.float32)

    @pl.when(pl.program_id(0) == 0)
    def _():
        o_ref[...] = jnp.zeros_like(o_ref)

    o_ref[...] += contrib


_W_SPEC = pl.BlockSpec((D, D), lambda i: (0, 0))
_B_SPEC = pl.BlockSpec((1, D), lambda i: (0, 0))
_ROW_SPEC = pl.BlockSpec((ROWS_BLK, D), lambda i: (i, 0))
_AGG_SPEC = pl.BlockSpec((2, ROWS_BLK, D), lambda i: (0, i, 0))

_proj_call = pl.pallas_call(
    _proj_body,
    grid=(GRID,),
    in_specs=[_ROW_SPEC, _W_SPEC],
    out_specs=_ROW_SPEC,
    out_shape=jax.ShapeDtypeStruct((N, D), jnp.float32),
)

_mlp_call = pl.pallas_call(
    _mlp_body,
    grid=(GRID,),
    in_specs=[_AGG_SPEC, _ROW_SPEC, _W_SPEC, _B_SPEC, _W_SPEC, _B_SPEC],
    out_specs=_ROW_SPEC,
    out_shape=jax.ShapeDtypeStruct((N, D), jnp.float32),
)

_final_call = pl.pallas_call(
    _final_body,
    grid=(GRID,),
    in_specs=[_AGG_SPEC, _ROW_SPEC, _W_SPEC, _B_SPEC, _W_SPEC, _B_SPEC,
              pl.BlockSpec((1, 1, ROWS_BLK), lambda i: (i, 0, 0))],
    out_specs=pl.BlockSpec((G, D), lambda i: (0, 0)),
    out_shape=jax.ShapeDtypeStruct((G, D), jnp.float32),
)


def kernel(x, edge_index, batch, W_proj, W1_0, b1_0, W2_0, b2_0,
           W1_1, b1_1, W2_1, b2_1):
    src = jnp.pad(edge_index[0], (0, EPAD - E))
    pad_dst = N + (jnp.arange(EPAD - E, dtype=jnp.int32) % (NPAD - N))
    dst = jnp.concatenate([edge_index[1], pad_dst])
    zeros = jnp.zeros((NPAD, D), jnp.float32)
    batch3d = batch.reshape(GRID, 1, ROWS_BLK)

    h = _proj_call(x, W_proj)

    agg = _agg_call(h, src, dst, zeros)
    h = _mlp_call(agg, h, W1_0, b1_0.reshape(1, D), W2_0, b2_0.reshape(1, D))

    agg = _agg_call(h, src, dst, zeros)
    out = _final_call(agg, h, W1_1, b1_1.reshape(1, D), W2_1,
                      b2_1.reshape(1, D), batch3d)
    return out


# speedup vs baseline: 2.2094x; 1.9751x over previous
import functools

import jax
import jax.numpy as jnp
from jax import lax
from jax.experimental import pallas as pl
from jax.experimental.pallas import tpu as pltpu
from jax.experimental.pallas import tpu_sc as plsc

N = 10000
E = 320000
D = 128
G = 128

NPAD = 10240
CH = 128
NW = 32
EPAD = 327680
NITER = EPAD // (NW * CH)
NPH = 2
CPH = NITER // NPH

ROWS_BLK = 1000
GRID = N // ROWS_BLK


def _agg_body(h_hbm, src_hbm, dst_hbm, zeros_hbm, out_hbm,
              src_v0, dst_v0, src_v1, dst_v1, rows_a, rows_b, acc_sh, sem_a, sem_b):
    cid = lax.axis_index("c")
    sid = lax.axis_index("s")
    wid = sid * 2 + cid

    pltpu.sync_copy(zeros_hbm.at[pl.ds(sid * 640, 640)],
                    acc_sh.at[pl.ds(sid * 640, 640)])
    plsc.subcore_barrier()

    def body(i, carry):
        k0 = (i * NW + wid) * CH
        pltpu.sync_copy(src_hbm.at[pl.ds(k0, CH)], src_v0)
        pltpu.sync_copy(dst_hbm.at[pl.ds(k0, CH)], dst_v0)
        pltpu.async_copy(h_hbm.at[src_v0], rows_a, sem_a).wait()
        pltpu.sync_copy(rows_a, acc_sh.at[dst_v0], add=True)
        return carry

    lax.fori_loop(0, NITER, body, 0)
    plsc.subcore_barrier()

    wbase = jnp.minimum(sid * 632, N - 632)
    pltpu.sync_copy(acc_sh.at[pl.ds(wbase, 632)],
                    out_hbm.at[cid, pl.ds(wbase, 632)])


_agg_call = pl.kernel(
    _agg_body,
    out_type=jax.ShapeDtypeStruct((2, N, D), jnp.float32),
    mesh=plsc.VectorSubcoreMesh(core_axis_name="c", subcore_axis_name="s"),
    scratch_types=[
        pltpu.VMEM((CH,), jnp.int32),
        pltpu.VMEM((CH,), jnp.int32),
        pltpu.VMEM((CH,), jnp.int32),
        pltpu.VMEM((CH,), jnp.int32),
        pltpu.VMEM((CH, D), jnp.float32),
        pltpu.VMEM((CH, D), jnp.float32),
        pltpu.VMEM_SHARED((NPAD, D), jnp.float32),
        pltpu.SemaphoreType.DMA,
        pltpu.SemaphoreType.DMA,
    ],
)


def _proj_body(x_ref, w_ref, o_ref):
    o_ref[...] = jnp.dot(x_ref[...], w_ref[...],
                         preferred_element_type=jnp.float32)


def _mlp_body(agg_ref, h_ref, w1_ref, b1_ref, w2_ref, b2_ref, o_ref):
    z = agg_ref[0] + agg_ref[1] + h_ref[...]
    z = jnp.maximum(
        jnp.dot(z, w1_ref[...], preferred_element_type=jnp.float32)
        + b1_ref[...], 0.0)
    z = jnp.maximum(
        jnp.dot(z, w2_ref[...], preferred_element_type=jnp.float32)
        + b2_ref[...], 0.0)
    o_ref[...] = z


def _final_body(agg_ref, h_ref, w1_ref, b1_ref, w2_ref, b2_ref, batch_ref,
                o_ref):
    z = agg_ref[0] + agg_ref[1] + h_ref[...]
    z = jnp.maximum(
        jnp.dot(z, w1_ref[...], preferred_element_type=jnp.float32)
        + b1_ref[...], 0.0)
    z = jnp.maximum(
        jnp.dot(z, w2_ref[...], preferred_element_type=jnp.float32)
        + b2_ref[...], 0.0)
    b = batch_ref[0, 0, :]
    onehot = (b[:, None]
              == lax.broadcasted_iota(jnp.int32, (ROWS_BLK, G), 1)
              ).astype(jnp.float32)
    contrib = lax.dot_general(onehot, z, (((0,), (0,)), ((), ())),
                              preferred_element_type=jnp.float32)

    @pl.when(pl.program_id(0) == 0)
    def _():
        o_ref[...] = jnp.zeros_like(o_ref)

    o_ref[...] += contrib


_W_SPEC = pl.BlockSpec((D, D), lambda i: (0, 0))
_B_SPEC = pl.BlockSpec((1, D), lambda i: (0, 0))
_ROW_SPEC = pl.BlockSpec((ROWS_BLK, D), lambda i: (i, 0))
_AGG_SPEC = pl.BlockSpec((2, ROWS_BLK, D), lambda i: (0, i, 0))

_proj_call = pl.pallas_call(
    _proj_body,
    grid=(GRID,),
    in_specs=[_ROW_SPEC, _W_SPEC],
    out_specs=_ROW_SPEC,
    out_shape=jax.ShapeDtypeStruct((N, D), jnp.float32),
)

_mlp_call = pl.pallas_call(
    _mlp_body,
    grid=(GRID,),
    in_specs=[_AGG_SPEC, _ROW_SPEC, _W_SPEC, _B_SPEC, _W_SPEC, _B_SPEC],
    out_specs=_ROW_SPEC,
    out_shape=jax.ShapeDtypeStruct((N, D), jnp.float32),
)

_final_call = pl.pallas_call(
    _final_body,
    grid=(GRID,),
    in_specs=[_AGG_SPEC, _ROW_SPEC, _W_SPEC, _B_SPEC, _W_SPEC, _B_SPEC,
              pl.BlockSpec((1, 1, ROWS_BLK), lambda i: (i, 0, 0))],
    out_specs=pl.BlockSpec((G, D), lambda i: (0, 0)),
    out_shape=jax.ShapeDtypeStruct((G, D), jnp.float32),
)


def kernel(x, edge_index, batch, W_proj, W1_0, b1_0, W2_0, b2_0,
           W1_1, b1_1, W2_1, b2_1):
    pad_iota = jnp.arange(EPAD - E, dtype=jnp.int32)
    src = jnp.concatenate([edge_index[0], pad_iota % N])
    dst = jnp.concatenate([edge_index[1], N + pad_iota % (NPAD - N)])
    zeros = jnp.zeros((NPAD, D), jnp.float32)
    batch3d = batch.reshape(GRID, 1, ROWS_BLK)

    h = _proj_call(x, W_proj)

    agg = _agg_call(h, src, dst, zeros)
    h = _mlp_call(agg, h, W1_0, b1_0.reshape(1, D), W2_0, b2_0.reshape(1, D))

    agg = _agg_call(h, src, dst, zeros)
    out = _final_call(agg, h, W1_1, b1_1.reshape(1, D), W2_1,
                      b2_1.reshape(1, D), batch3d)
    return out


# speedup vs baseline: 3.3739x; 1.5271x over previous
import functools

import jax
import jax.numpy as jnp
from jax import lax
from jax.experimental import pallas as pl
from jax.experimental.pallas import tpu as pltpu
from jax.experimental.pallas import tpu_sc as plsc

N = 10000
E = 320000
D = 128
G = 128

NPAD = 10240
CH = 128
NW = 32
EPAD = 327680
NITER = EPAD // (NW * CH)
NPH = 2
CPH = NITER // NPH

ROWS_BLK = 1000
GRID = N // ROWS_BLK


def _agg_body(h_hbm, src_hbm, dst_hbm, zeros_hbm, out_hbm,
              src_v0, dst_v0, src_v1, dst_v1, rows_a, rows_b, acc_sh, sem_a, sem_b):
    cid = lax.axis_index("c")
    sid = lax.axis_index("s")
    wid = sid * 2 + cid

    pltpu.sync_copy(zeros_hbm.at[pl.ds(sid * 640, 640)],
                    acc_sh.at[pl.ds(sid * 640, 640)])
    plsc.subcore_barrier()

    base0 = wid * NITER * CH

    pltpu.sync_copy(src_hbm.at[pl.ds(base0, CH)], src_v0)
    pltpu.sync_copy(dst_hbm.at[pl.ds(base0, CH)], dst_v0)
    pltpu.async_copy(h_hbm.at[src_v0], rows_a, sem_a)

    def body(i, carry):
        k1 = base0 + (2 * i + 1) * CH
        k2 = base0 + jnp.minimum(2 * i + 2, NITER - 1) * CH
        pltpu.sync_copy(src_hbm.at[pl.ds(k1, CH)], src_v1)
        pltpu.sync_copy(dst_hbm.at[pl.ds(k1, CH)], dst_v1)
        pltpu.async_copy(h_hbm.at[src_v1], rows_b, sem_b)
        pltpu.make_async_copy(h_hbm.at[src_v0], rows_a, sem_a).wait()
        pltpu.sync_copy(rows_a, acc_sh.at[dst_v0], add=True)
        pltpu.sync_copy(src_hbm.at[pl.ds(k2, CH)], src_v0)
        pltpu.sync_copy(dst_hbm.at[pl.ds(k2, CH)], dst_v0)
        pltpu.async_copy(h_hbm.at[src_v0], rows_a, sem_a)
        pltpu.make_async_copy(h_hbm.at[src_v1], rows_b, sem_b).wait()
        pltpu.sync_copy(rows_b, acc_sh.at[dst_v1], add=True)
        return carry

    lax.fori_loop(0, NITER // 2, body, 0)
    pltpu.make_async_copy(h_hbm.at[src_v0], rows_a, sem_a).wait()
    plsc.subcore_barrier()

    wbase = jnp.minimum(sid * 632, N - 632)
    pltpu.sync_copy(acc_sh.at[pl.ds(wbase, 632)],
                    out_hbm.at[cid, pl.ds(wbase, 632)])


_agg_call = pl.kernel(
    _agg_body,
    out_type=jax.ShapeDtypeStruct((2, N, D), jnp.float32),
    mesh=plsc.VectorSubcoreMesh(core_axis_name="c", subcore_axis_name="s"),
    scratch_types=[
        pltpu.VMEM((CH,), jnp.int32),
        pltpu.VMEM((CH,), jnp.int32),
        pltpu.VMEM((CH,), jnp.int32),
        pltpu.VMEM((CH,), jnp.int32),
        pltpu.VMEM((CH, D), jnp.float32),
        pltpu.VMEM((CH, D), jnp.float32),
        pltpu.VMEM_SHARED((NPAD, D), jnp.float32),
        pltpu.SemaphoreType.DMA,
        pltpu.SemaphoreType.DMA,
    ],
)


def _proj_body(x_ref, w_ref, o_ref):
    o_ref[...] = jnp.dot(x_ref[...], w_ref[...],
                         preferred_element_type=jnp.float32)


def _mlp_body(agg_ref, h_ref, w1_ref, b1_ref, w2_ref, b2_ref, o_ref):
    z = agg_ref[0] + agg_ref[1] + h_ref[...]
    z = jnp.maximum(
        jnp.dot(z, w1_ref[...], preferred_element_type=jnp.float32)
        + b1_ref[...], 0.0)
    z = jnp.maximum(
        jnp.dot(z, w2_ref[...], preferred_element_type=jnp.float32)
        + b2_ref[...], 0.0)
    o_ref[...] = z


def _final_body(agg_ref, h_ref, w1_ref, b1_ref, w2_ref, b2_ref, batch_ref,
                o_ref):
    z = agg_ref[0] + agg_ref[1] + h_ref[...]
    z = jnp.maximum(
        jnp.dot(z, w1_ref[...], preferred_element_type=jnp.float32)
        + b1_ref[...], 0.0)
    z = jnp.maximum(
        jnp.dot(z, w2_ref[...], preferred_element_type=jnp.float32)
        + b2_ref[...], 0.0)
    b = batch_ref[0, 0, :]
    onehot = (b[:, None]
              == lax.broadcasted_iota(jnp.int32, (ROWS_BLK, G), 1)
              ).astype(jnp.float32)
    contrib = lax.dot_general(onehot, z, (((0,), (0,)), ((), ())),
                              preferred_element_type=jnp.float32)

    @pl.when(pl.program_id(0) == 0)
    def _():
        o_ref[...] = jnp.zeros_like(o_ref)

    o_ref[...] += contrib


_W_SPEC = pl.BlockSpec((D, D), lambda i: (0, 0))
_B_SPEC = pl.BlockSpec((1, D), lambda i: (0, 0))
_ROW_SPEC = pl.BlockSpec((ROWS_BLK, D), lambda i: (i, 0))
_AGG_SPEC = pl.BlockSpec((2, ROWS_BLK, D), lambda i: (0, i, 0))

_proj_call = pl.pallas_call(
    _proj_body,
    grid=(GRID,),
    in_specs=[_ROW_SPEC, _W_SPEC],
    out_specs=_ROW_SPEC,
    out_shape=jax.ShapeDtypeStruct((N, D), jnp.float32),
)

_mlp_call = pl.pallas_call(
    _mlp_body,
    grid=(GRID,),
    in_specs=[_AGG_SPEC, _ROW_SPEC, _W_SPEC, _B_SPEC, _W_SPEC, _B_SPEC],
    out_specs=_ROW_SPEC,
    out_shape=jax.ShapeDtypeStruct((N, D), jnp.float32),
)

_final_call = pl.pallas_call(
    _final_body,
    grid=(GRID,),
    in_specs=[_AGG_SPEC, _ROW_SPEC, _W_SPEC, _B_SPEC, _W_SPEC, _B_SPEC,
              pl.BlockSpec((1, 1, ROWS_BLK), lambda i: (i, 0, 0))],
    out_specs=pl.BlockSpec((G, D), lambda i: (0, 0)),
    out_shape=jax.ShapeDtypeStruct((G, D), jnp.float32),
)


def kernel(x, edge_index, batch, W_proj, W1_0, b1_0, W2_0, b2_0,
           W1_1, b1_1, W2_1, b2_1):
    pad_iota = jnp.arange(EPAD - E, dtype=jnp.int32)
    src = jnp.concatenate([edge_index[0], pad_iota % N])
    dst = jnp.concatenate([edge_index[1], N + pad_iota % (NPAD - N)])
    zeros = jnp.zeros((NPAD, D), jnp.float32)
    batch3d = batch.reshape(GRID, 1, ROWS_BLK)

    h = _proj_call(x, W_proj)

    agg = _agg_call(h, src, dst, zeros)
    h = _mlp_call(agg, h, W1_0, b1_0.reshape(1, D), W2_0, b2_0.reshape(1, D))

    agg = _agg_call(h, src, dst, zeros)
    out = _final_call(agg, h, W1_1, b1_1.reshape(1, D), W2_1,
                      b2_1.reshape(1, D), batch3d)
    return out
